# baseline (speedup 1.0000x reference)
import functools
from itertools import combinations

import numpy as np
import jax
import jax.numpy as jnp
from jax import lax
from jax.experimental import pallas as pl
from jax.experimental.pallas import tpu as pltpu
from jax.experimental.pallas import tpu_sc as plsc

_SIZE_IN = 14
_HOTNESS = 3
_BATCH = 4096
_IN_DIM = 2 ** _SIZE_IN
_NCOL = 364
_LN = 16

_NW = 32
_RSC = 2048
_BR = 512
_RB = 8
_TROWS_PER_W = (_RSC // _RB) // _NW
_NTILE = 64
_NG = 23


def _build_constants():
    idx = np.array(
        [sum(2 ** i for i in ones) for ones in combinations(range(_SIZE_IN), _HOTNESS)],
        dtype=np.int32,
    )
    tiles = np.unique(idx >> 7)
    assert len(tiles) == _NTILE
    pos = {int(t): p for p, t in enumerate(tiles)}

    spans = []
    s = prev = int(tiles[0])
    for t in tiles[1:]:
        t = int(t)
        if t == prev + 1:
            prev = t
        else:
            spans.append((s, pos[s], prev - s + 1))
            s = prev = t
    spans.append((s, pos[s], prev - s + 1))

    gstart = [g * _LN for g in range(_NCOL // _LN)] + [_NCOL - _LN]
    selc = np.zeros(_NG * _LN, np.int32)
    for g, st in enumerate(gstart):
        for t in range(_LN):
            v = int(idx[st + t])
            selc[g * _LN + t] = pos[v >> 7] * 128 + (v & 127)
    return gstart, spans, selc


_GSTART, _SPANS, _SELC = _build_constants()

_mesh = plsc.VectorSubcoreMesh(core_axis_name="c", subcore_axis_name="s")


@functools.partial(
    pl.kernel,
    out_type=jax.ShapeDtypeStruct((_RSC, _NCOL), jnp.float32),
    mesh=_mesh,
    compiler_params=pltpu.CompilerParams(needs_layout_passes=False,
                                         use_tc_tiling_on_sc=True),
    scratch_types=[
        pltpu.VMEM((_RB // 2, _NTILE * 128), jnp.float32),
        pltpu.VMEM((_RB // 2, _NTILE * 128), jnp.float32),
        pltpu.VMEM((_RB, _NCOL), jnp.float32),
        pltpu.VMEM((_NG * _LN,), jnp.int32),
        pltpu.SemaphoreType.DMA,
        pltpu.SemaphoreType.DMA,
    ],
)
def _gather_affine(table, selc_hbm, out, st_a, st_b, out_v, selc_v,
                   sem_a, sem_b):
    wid = lax.axis_index("s") * 2 + lax.axis_index("c")
    pltpu.sync_copy(selc_hbm, selc_v)
    tr0 = wid * _TROWS_PER_W
    hb = _RB // 2

    def issue(row0, st, sem):
        for (ts, ps, n) in _SPANS:
            pltpu.async_copy(
                table.at[pl.ds(row0, hb), pl.ds(ts * 128, n * 128)],
                st.at[:, pl.ds(ps * 128, n * 128)], sem)

    def drain(st, sem):
        pltpu.make_async_copy(
            table.at[pl.ds(0, hb), pl.ds(0, _NTILE * 128)], st, sem).wait()

    def select(st, rbase):
        for g in range(_NG):
            cc = selc_v[pl.ds(g * _LN, _LN)]
            base = _GSTART[g]
            for r in range(hb):
                rr = jnp.full((_LN,), r, jnp.int32)
                v = plsc.load_gather(st, [rr, cc])
                out_v[rbase + r, pl.ds(base, _LN)] = v * 12.0 - 6.0

    issue(tr0 * _RB, st_a, sem_a)

    def blk_body(b, carry):
        row0 = (tr0 + b) * _RB
        issue(row0 + hb, st_b, sem_b)
        drain(st_a, sem_a)
        select(st_a, 0)
        nxt = tr0 * _RB + lax.rem(row0 + _RB - tr0 * _RB,
                                  _TROWS_PER_W * _RB)
        issue(nxt, st_a, sem_a)
        drain(st_b, sem_b)
        select(st_b, hb)
        pltpu.sync_copy(out_v, out.at[pl.ds(row0, _RB)])
        return carry

    lax.fori_loop(0, _TROWS_PER_W, blk_body, 0)
    drain(st_a, sem_a)


def _build_onehot():
    idx = np.array(
        [sum(2 ** i for i in ones) for ones in combinations(range(_SIZE_IN), _HOTNESS)],
        dtype=np.int32,
    )
    tiles = np.unique(idx >> 7)
    pos = {int(t): p for p, t in enumerate(tiles)}
    sel = np.zeros((_NTILE * 128, _NCOL), np.float32)
    for j, v in enumerate(idx):
        sel[pos[int(v) >> 7] * 128 + (int(v) & 127), j] = 1.0
    tl = [int(t) for t in tiles]
    return np.asarray(tl + [tl[-1]], np.int32), sel


_TILES_ARR, _ONEHOT = _build_onehot()


def _tc_body(s_ref, x_ref, sel_ref, o_ref, xsel):
    j = pl.program_id(1)

    @pl.when(j < _NTILE)
    def _stage():
        xsel[:, pl.ds(j * 128, 128)] = x_ref[...].astype(jnp.bfloat16)

    @pl.when(j == _NTILE)
    def _fin():
        acc = jax.lax.dot_general(
            xsel[...], sel_ref[...],
            (((1,), (0,)), ((), ())), preferred_element_type=jnp.float32)
        o_ref[...] = acc * 12.0 - 6.0


_tc_select = pl.pallas_call(
    _tc_body,
    grid_spec=pltpu.PrefetchScalarGridSpec(
        num_scalar_prefetch=1,
        grid=((_BATCH - _RSC) // _BR, _NTILE + 1),
        in_specs=[
            pl.BlockSpec((_BR, 128), lambda i, j, s: (i + _RSC // _BR, s[j])),
            pl.BlockSpec((_NTILE * 128, _NCOL), lambda i, j, s: (0, 0)),
        ],
        out_specs=pl.BlockSpec((_BR, _NCOL), lambda i, j, s: (i, 0)),
        scratch_shapes=[pltpu.VMEM((_BR, _NTILE * 128), jnp.bfloat16)],
    ),
    out_shape=jax.ShapeDtypeStruct((_BATCH - _RSC, _NCOL), jnp.float32),
    compiler_params=pltpu.CompilerParams(
        dimension_semantics=("parallel", "arbitrary")),
)


def kernel(input_var):
    out_sc = _gather_affine(input_var, jnp.asarray(_SELC))
    out_tc = _tc_select(jnp.asarray(_TILES_ARR), input_var,
                        jnp.asarray(_ONEHOT, jnp.bfloat16))
    return jnp.concatenate([out_sc, out_tc], axis=0)


# speedup vs baseline: 2.2213x; 2.2213x over previous
import functools
from itertools import combinations

import numpy as np
import jax
import jax.numpy as jnp
from jax import lax
from jax.experimental import pallas as pl
from jax.experimental.pallas import tpu as pltpu
from jax.experimental.pallas import tpu_sc as plsc

_SIZE_IN = 14
_HOTNESS = 3
_BATCH = 4096
_IN_DIM = 2 ** _SIZE_IN
_NCOL = 364
_LN = 16

_NW = 32
_RB = 8
_TROWS_PER_W = (_BATCH // _RB) // _NW
_NTILE = 64
_NG = 23


def _build_constants():
    idx = np.array(
        [sum(2 ** i for i in ones) for ones in combinations(range(_SIZE_IN), _HOTNESS)],
        dtype=np.int32,
    )
    tiles = np.unique(idx >> 7)
    assert len(tiles) == _NTILE
    pos = {int(t): p for p, t in enumerate(tiles)}

    spans = []
    s = prev = int(tiles[0])
    for t in tiles[1:]:
        t = int(t)
        if t == prev + 1:
            prev = t
        else:
            spans.append((s, pos[s], prev - s + 1))
            s = prev = t
    spans.append((s, pos[s], prev - s + 1))

    gstart = [g * _LN for g in range(_NCOL // _LN)] + [_NCOL - _LN]
    selc = np.zeros(_NG * _LN, np.int32)
    for g, st in enumerate(gstart):
        for t in range(_LN):
            v = int(idx[st + t])
            selc[g * _LN + t] = pos[v >> 7] * 128 + (v & 127)
    return gstart, spans, selc


_GSTART, _SPANS, _SELC = _build_constants()

_mesh = plsc.VectorSubcoreMesh(core_axis_name="c", subcore_axis_name="s")


@functools.partial(
    pl.kernel,
    out_type=jax.ShapeDtypeStruct((_BATCH, _NCOL), jnp.float32),
    mesh=_mesh,
    compiler_params=pltpu.CompilerParams(needs_layout_passes=False,
                                         use_tc_tiling_on_sc=True),
    scratch_types=[
        pltpu.VMEM((_RB // 2, _NTILE * 128), jnp.float32),
        pltpu.VMEM((_RB // 2, _NTILE * 128), jnp.float32),
        pltpu.VMEM((_RB, _NCOL), jnp.float32),
        pltpu.VMEM((_NG * _LN,), jnp.int32),
        pltpu.SemaphoreType.DMA,
        pltpu.SemaphoreType.DMA,
    ],
)
def _gather_affine(table, selc_hbm, out, st_a, st_b, out_v, selc_v,
                   sem_a, sem_b):
    wid = lax.axis_index("s") * 2 + lax.axis_index("c")
    pltpu.sync_copy(selc_hbm, selc_v)
    tr0 = wid * _TROWS_PER_W
    hb = _RB // 2

    def issue(row0, st, sem):
        for (ts, ps, n) in _SPANS:
            pltpu.async_copy(
                table.at[pl.ds(row0, hb), pl.ds(ts * 128, n * 128)],
                st.at[:, pl.ds(ps * 128, n * 128)], sem)

    def drain(st, sem):
        pltpu.make_async_copy(
            table.at[pl.ds(0, hb), pl.ds(0, _NTILE * 128)], st, sem).wait()

    def select(st, rbase):
        for g in range(_NG):
            cc = selc_v[pl.ds(g * _LN, _LN)]
            base = _GSTART[g]
            for r in range(hb):
                rr = jnp.full((_LN,), r, jnp.int32)
                v = plsc.load_gather(st, [rr, cc])
                out_v[rbase + r, pl.ds(base, _LN)] = v * 12.0 - 6.0

    issue(tr0 * _RB, st_a, sem_a)

    def blk_body(b, carry):
        row0 = (tr0 + b) * _RB
        issue(row0 + hb, st_b, sem_b)
        drain(st_a, sem_a)
        select(st_a, 0)
        nxt = tr0 * _RB + lax.rem(row0 + _RB - tr0 * _RB,
                                  _TROWS_PER_W * _RB)
        issue(nxt, st_a, sem_a)
        drain(st_b, sem_b)
        select(st_b, hb)
        pltpu.sync_copy(out_v, out.at[pl.ds(row0, _RB)])
        return carry

    lax.fori_loop(0, _TROWS_PER_W, blk_body, 0)
    drain(st_a, sem_a)


def kernel(input_var):
    return _gather_affine(input_var, jnp.asarray(_SELC))
